# baseline (speedup 1.0000x reference)
import functools

import jax
import jax.numpy as jnp
from jax import lax
from jax.experimental import pallas as pl
from jax.experimental.pallas import tpu as pltpu
from jax.experimental.pallas import tpu_sc as plsc

B = 512
L = 100
NNODES = 1000
LOOKBACK = 30
NPAD = 1024
LPAD = 128
NC = 2
NS = 16
LANES = 16
NWORK = NC * NS
SPW = B // NWORK

NCHUNKS = 63
TAIL_VALID = NNODES - (NCHUNKS - 1) * LANES
LN2 = 0.6931471805599453
SQRT2 = 1.4142135623730951


def _proj_body(x_ref, w_ref, dist_ref, wv_ref, yk_ref, l0_ref):
    xb = x_ref[...]
    w = w_ref[...]
    wl = w[:, 0].reshape(1, LOOKBACK, 1)
    wr = w[:, 1].reshape(1, LOOKBACK, 1)
    y = jnp.sum(xb * wl, axis=1)
    z = jnp.sum(xb * wr, axis=1)
    w_out = wv_ref[0]
    w_dist = wv_ref[1]
    yk = w_out * y
    l0 = w_out * z + w_dist * dist_ref[...]
    pad = jnp.zeros((y.shape[0], NPAD - NNODES), jnp.float32)
    yk_ref[...] = jnp.concatenate([yk, pad], axis=1)
    l0_ref[...] = jnp.concatenate([l0, pad], axis=1)


def _project(x, w, dist2d, wv):
    Bb = 64
    return pl.pallas_call(
        _proj_body,
        grid=(B // Bb,),
        in_specs=[
            pl.BlockSpec((Bb, LOOKBACK, NNODES), lambda b: (b, 0, 0)),
            pl.BlockSpec((LOOKBACK, 2), lambda b: (0, 0)),
            pl.BlockSpec((1, NNODES), lambda b: (0, 0)),
            pl.BlockSpec(memory_space=pltpu.SMEM),
        ],
        out_specs=[
            pl.BlockSpec((Bb, NPAD), lambda b: (b, 0)),
            pl.BlockSpec((Bb, NPAD), lambda b: (b, 0)),
        ],
        out_shape=[
            jax.ShapeDtypeStruct((B, NPAD), jnp.float32),
            jax.ShapeDtypeStruct((B, NPAD), jnp.float32),
        ],
    )(x, w, dist2d, wv)


def _vlog(v):
    bits = plsc.bitcast(v, jnp.int32)
    e = jnp.right_shift(bits, 23) - 127
    mant = jnp.bitwise_or(jnp.bitwise_and(bits, 0x7FFFFF), 127 << 23)
    m = plsc.bitcast(mant, jnp.float32)
    big = m > SQRT2
    m = jnp.where(big, m * 0.5, m)
    ef = (e + jnp.where(big, 1, 0)).astype(jnp.float32)
    r = (m - 1.0) / (m + 1.0)
    r2 = r * r
    ln_m = 2.0 * r * (1.0 + r2 * (1.0 / 3.0 + r2 * (0.2 + r2 * (1.0 / 7.0 + r2 / 9.0))))
    return ln_m + ef * LN2


def _sample_fused(agg_out_hbm, stops_v, y_v, l0_v, mask_v, sums_v, cnt_v, gv,
                  sem_out, i):
    zero16f = jnp.zeros((LANES,), jnp.float32)
    ones16f = jnp.ones((LANES,), jnp.float32)
    one16f = jnp.full((LANES,), 1.0, jnp.float32)
    lane = lax.broadcasted_iota(jnp.int32, (LANES,), 0)
    edge_tail_mask = lane < ((L - 1) - 6 * LANES)
    node_tail = lane < TAIL_VALID

    for k in range(NCHUNKS):
        sums_v[pl.ds(k * LANES, LANES)] = zero16f
        cnt_v[pl.ds(k * LANES, LANES)] = zero16f
    for c in range(7):
        idx = stops_v[pl.ds(c * LANES, LANES)]
        gv[pl.ds(c * LANES, LANES)] = plsc.load_gather(y_v, [idx])
    for c in range(7):
        didx = stops_v[pl.ds(c * LANES + 1, LANES)]
        vals = gv[pl.ds(c * LANES, LANES)]
        if c < 6:
            plsc.addupdate_scatter(sums_v, [didx], vals)
            plsc.addupdate_scatter(cnt_v, [didx], ones16f)
        else:
            plsc.addupdate_scatter(sums_v, [didx], vals, mask=edge_tail_mask)
            plsc.addupdate_scatter(cnt_v, [didx], ones16f, mask=edge_tail_mask)
    m16 = jnp.full((LANES,), -1e30, jnp.float32)
    for k in range(NCHUNKS):
        s = sums_v[pl.ds(k * LANES, LANES)]
        cc = cnt_v[pl.ds(k * LANES, LANES)]
        lg = s / jnp.maximum(cc, one16f) + l0_v[pl.ds(k * LANES, LANES)]
        sums_v[pl.ds(k * LANES, LANES)] = lg
        if k == NCHUNKS - 1:
            lg = jnp.where(node_tail, lg, -1e30)
        m16 = jnp.maximum(m16, lg)
    m = jnp.max(m16)
    msplat = jnp.full((LANES,), 1.0, jnp.float32) * m
    s16 = jnp.zeros((LANES,), jnp.float32)
    for k in range(NCHUNKS):
        ex = jnp.exp(sums_v[pl.ds(k * LANES, LANES)] - msplat)
        if k == NCHUNKS - 1:
            ex = jnp.where(node_tail, ex, zero16f)
        s16 = s16 + ex
    lse = _vlog(jnp.full((LANES,), 1.0, jnp.float32) * jnp.sum(s16)) + msplat
    neg16 = jnp.full((LANES,), -1e8, jnp.float32)
    for k in range(NCHUNKS):
        logp = sums_v[pl.ds(k * LANES, LANES)] - lse
        mk = mask_v[pl.ds(k * LANES, LANES)]
        sums_v[pl.ds(k * LANES, LANES)] = jnp.where(mk != 0, neg16, logp)
    pltpu.async_copy(sums_v.at[pl.ds(0, NNODES)],
                     agg_out_hbm.at[pl.ds(i * NNODES, NNODES)], sem_out)


def _seg_softmax_body(stops_hbm, y_hbm, l0_hbm, mask_hbm, out_hbm,
                      stops_v0, stops_v1, y_v0, y_v1, l0_v0, l0_v1,
                      mask_v0, mask_v1, sums_v0, sums_v1, cnt_v, gv,
                      sem_in0, sem_in1, sem_out0, sem_out1):
    cid = lax.axis_index("c")
    sid = lax.axis_index("s")
    wid = sid * NC + cid
    base = wid * SPW

    def start_in(i, sv, yv, lv, mv, sem):
        pltpu.async_copy(stops_hbm.at[i], sv, sem)
        pltpu.async_copy(y_hbm.at[i], yv, sem)
        pltpu.async_copy(l0_hbm.at[i], lv, sem)
        pltpu.async_copy(mask_hbm.at[pl.ds(i * NNODES, NNODES)],
                         mv.at[pl.ds(0, NNODES)], sem)

    def wait_in(i, sv, yv, lv, mv, sem):
        pltpu.make_async_copy(stops_hbm.at[i], sv, sem).wait()
        pltpu.make_async_copy(y_hbm.at[i], yv, sem).wait()
        pltpu.make_async_copy(l0_hbm.at[i], lv, sem).wait()
        pltpu.make_async_copy(mask_hbm.at[pl.ds(i * NNODES, NNODES)],
                              mv.at[pl.ds(0, NNODES)], sem).wait()

    def wait_out(sums_v, sem):
        pltpu.make_async_copy(sums_v.at[pl.ds(0, NNODES)],
                              out_hbm.at[pl.ds(base * NNODES, NNODES)], sem).wait()

    start_in(base, stops_v0, y_v0, l0_v0, mask_v0, sem_in0)

    def body(j2, carry):
        i0 = base + 2 * j2
        i1 = i0 + 1
        wait_in(i0, stops_v0, y_v0, l0_v0, mask_v0, sem_in0)
        start_in(i1, stops_v1, y_v1, l0_v1, mask_v1, sem_in1)

        @pl.when(j2 >= 1)
        def _():
            wait_out(sums_v0, sem_out0)

        _sample_fused(out_hbm, stops_v0, y_v0, l0_v0, mask_v0, sums_v0, cnt_v,
                      gv, sem_out0, i0)
        wait_in(i1, stops_v1, y_v1, l0_v1, mask_v1, sem_in1)

        @pl.when(2 * j2 + 2 < SPW)
        def _():
            start_in(i0 + 2, stops_v0, y_v0, l0_v0, mask_v0, sem_in0)

        @pl.when(j2 >= 1)
        def _():
            wait_out(sums_v1, sem_out1)

        _sample_fused(out_hbm, stops_v1, y_v1, l0_v1, mask_v1, sums_v1, cnt_v,
                      gv, sem_out1, i1)
        return carry

    lax.fori_loop(0, SPW // 2, body, 0)
    wait_out(sums_v0, sem_out0)
    wait_out(sums_v1, sem_out1)


def _seg_softmax(stops_padded, yk, l0, x_mask):
    mesh = plsc.VectorSubcoreMesh(
        core_axis_name="c", subcore_axis_name="s", num_cores=NC, num_subcores=NS
    )
    f = pl.kernel(
        _seg_softmax_body,
        out_type=jax.ShapeDtypeStruct((B * NNODES,), jnp.float32),
        mesh=mesh,
        scratch_types=[
            pltpu.VMEM((LPAD,), jnp.int32),
            pltpu.VMEM((LPAD,), jnp.int32),
            pltpu.VMEM((NPAD,), jnp.float32),
            pltpu.VMEM((NPAD,), jnp.float32),
            pltpu.VMEM((NPAD,), jnp.float32),
            pltpu.VMEM((NPAD,), jnp.float32),
            pltpu.VMEM((NPAD,), jnp.int32),
            pltpu.VMEM((NPAD,), jnp.int32),
            pltpu.VMEM((NPAD,), jnp.float32),
            pltpu.VMEM((NPAD,), jnp.float32),
            pltpu.VMEM((NPAD,), jnp.float32),
            pltpu.VMEM((7 * LANES,), jnp.float32),
            pltpu.SemaphoreType.DMA,
            pltpu.SemaphoreType.DMA,
            pltpu.SemaphoreType.DMA,
            pltpu.SemaphoreType.DMA,
        ],
        compiler_params=pltpu.CompilerParams(needs_layout_passes=False),
    )
    return f(stops_padded, yk, l0, x_mask.reshape(B * NNODES)).reshape(B, NNODES)


def kernel(stops, x, x_dist, x_features, x_week, x_mask, stop_emb_table,
           week_emb_table, W_l, b_l, W_r, W_fc2, b_fc2):
    w = jnp.concatenate([W_l, W_r], axis=1)
    wv = jnp.stack([W_fc2[78, 0], W_fc2[79, 0]])
    dist2d = x_dist.reshape(1, NNODES)
    yk, l0 = _project(x, w, dist2d, wv)
    stops_padded = jnp.pad(stops, ((0, 0), (0, LPAD - L)))
    return _seg_softmax(stops_padded, yk, l0, x_mask)


# speedup vs baseline: 1.0019x; 1.0019x over previous
import functools

import jax
import jax.numpy as jnp
from jax import lax
from jax.experimental import pallas as pl
from jax.experimental.pallas import tpu as pltpu
from jax.experimental.pallas import tpu_sc as plsc

B = 512
L = 100
NNODES = 1000
LOOKBACK = 30
NPAD = 1024
LPAD = 128
NC = 2
NS = 16
LANES = 16
NWORK = NC * NS
SPW = B // NWORK

NCHUNKS = 63
TAIL_VALID = NNODES - (NCHUNKS - 1) * LANES
LN2 = 0.6931471805599453
SQRT2 = 1.4142135623730951


def _proj_body(x_ref, w_ref, dist_ref, wv_ref, yk_ref, l0_ref):
    xb = x_ref[...]
    w = w_ref[...]
    wl = w[:, 0].reshape(1, LOOKBACK, 1)
    wr = w[:, 1].reshape(1, LOOKBACK, 1)
    y = jnp.sum(xb * wl, axis=1)
    z = jnp.sum(xb * wr, axis=1)
    w_out = wv_ref[0]
    w_dist = wv_ref[1]
    yk = w_out * y
    l0 = w_out * z + w_dist * dist_ref[...]
    pad = jnp.zeros((y.shape[0], NPAD - NNODES), jnp.float32)
    yk_ref[...] = jnp.concatenate([yk, pad], axis=1)
    l0_ref[...] = jnp.concatenate([l0, pad], axis=1)


def _project(x, w, dist2d, wv):
    Bb = 64
    return pl.pallas_call(
        _proj_body,
        grid=(B // Bb,),
        in_specs=[
            pl.BlockSpec((Bb, LOOKBACK, NNODES), lambda b: (b, 0, 0)),
            pl.BlockSpec((LOOKBACK, 2), lambda b: (0, 0)),
            pl.BlockSpec((1, NNODES), lambda b: (0, 0)),
            pl.BlockSpec(memory_space=pltpu.SMEM),
        ],
        out_specs=[
            pl.BlockSpec((Bb, NPAD), lambda b: (b, 0)),
            pl.BlockSpec((Bb, NPAD), lambda b: (b, 0)),
        ],
        out_shape=[
            jax.ShapeDtypeStruct((B, NPAD), jnp.float32),
            jax.ShapeDtypeStruct((B, NPAD), jnp.float32),
        ],
    )(x, w, dist2d, wv)


def _vlog(v):
    bits = plsc.bitcast(v, jnp.int32)
    e = jnp.right_shift(bits, 23) - 127
    mant = jnp.bitwise_or(jnp.bitwise_and(bits, 0x7FFFFF), 127 << 23)
    m = plsc.bitcast(mant, jnp.float32)
    big = m > SQRT2
    m = jnp.where(big, m * 0.5, m)
    ef = (e + jnp.where(big, 1, 0)).astype(jnp.float32)
    r = (m - 1.0) / (m + 1.0)
    r2 = r * r
    ln_m = 2.0 * r * (1.0 + r2 * (1.0 / 3.0 + r2 * (0.2 + r2 * (1.0 / 7.0 + r2 / 9.0))))
    return ln_m + ef * LN2


def _sample_fused(out_hbm, stops_v, y_v, l0_v, mask_v, sums_v, cnt_v, out_v,
                  sem_out, i):
    zero16f = jnp.zeros((LANES,), jnp.float32)
    ones16f = jnp.ones((LANES,), jnp.float32)
    one16f = jnp.full((LANES,), 1.0, jnp.float32)
    lane = lax.broadcasted_iota(jnp.int32, (LANES,), 0)
    edge_tail = lane < ((L - 1) - 6 * LANES)
    node_tail = lane < TAIL_VALID

    didx = [stops_v[pl.ds(c * LANES + 1, LANES)] for c in range(7)]
    emask = [None] * 6 + [edge_tail]
    for c in range(7):
        plsc.store_scatter(sums_v, [didx[c]], zero16f, mask=emask[c])
        plsc.store_scatter(cnt_v, [didx[c]], zero16f, mask=emask[c])
    for c in range(7):
        sidx = stops_v[pl.ds(c * LANES, LANES)]
        vals = plsc.load_gather(y_v, [sidx])
        plsc.addupdate_scatter(sums_v, [didx[c]], vals, mask=emask[c])
        plsc.addupdate_scatter(cnt_v, [didx[c]], ones16f, mask=emask[c])
    lg = []
    for c in range(7):
        sc_ = plsc.load_gather(sums_v, [didx[c]])
        cc = plsc.load_gather(cnt_v, [didx[c]])
        lo = plsc.load_gather(l0_v, [didx[c]])
        lg.append(sc_ / jnp.maximum(cc, one16f) + lo)
    for c in range(7):
        plsc.store_scatter(l0_v, [didx[c]], lg[c], mask=emask[c])
    m16 = jnp.full((LANES,), -1e30, jnp.float32)
    for k in range(NCHUNKS):
        v = l0_v[pl.ds(k * LANES, LANES)]
        if k == NCHUNKS - 1:
            v = jnp.where(node_tail, v, -1e30)
        m16 = jnp.maximum(m16, v)
    m = jnp.max(m16)
    msplat = jnp.full((LANES,), 1.0, jnp.float32) * m
    s16 = jnp.zeros((LANES,), jnp.float32)
    for k in range(NCHUNKS):
        ex = jnp.exp(l0_v[pl.ds(k * LANES, LANES)] - msplat)
        if k == NCHUNKS - 1:
            ex = jnp.where(node_tail, ex, zero16f)
        s16 = s16 + ex
    lse = _vlog(jnp.full((LANES,), 1.0, jnp.float32) * jnp.sum(s16)) + msplat
    neg16 = jnp.full((LANES,), -1e8, jnp.float32)
    for k in range(NCHUNKS):
        logp = l0_v[pl.ds(k * LANES, LANES)] - lse
        mk = mask_v[pl.ds(k * LANES, LANES)]
        out_v[pl.ds(k * LANES, LANES)] = jnp.where(mk != 0, neg16, logp)
    pltpu.async_copy(out_v.at[pl.ds(0, NNODES)],
                     out_hbm.at[pl.ds(i * NNODES, NNODES)], sem_out)


def _seg_softmax_body(stops_hbm, y_hbm, l0_hbm, mask_hbm, out_hbm,
                      stops_v0, stops_v1, y_v0, y_v1, l0_v0, l0_v1,
                      mask_v0, mask_v1, out_v0, out_v1, sums_v, cnt_v,
                      sem_in0, sem_in1, sem_out0, sem_out1):
    cid = lax.axis_index("c")
    sid = lax.axis_index("s")
    wid = sid * NC + cid
    base = wid * SPW

    def start_in(i, sv, yv, lv, mv, sem):
        pltpu.async_copy(stops_hbm.at[i], sv, sem)
        pltpu.async_copy(y_hbm.at[i], yv, sem)
        pltpu.async_copy(l0_hbm.at[i], lv, sem)
        pltpu.async_copy(mask_hbm.at[pl.ds(i * NNODES, NNODES)],
                         mv.at[pl.ds(0, NNODES)], sem)

    def wait_in(i, sv, yv, lv, mv, sem):
        pltpu.make_async_copy(stops_hbm.at[i], sv, sem).wait()
        pltpu.make_async_copy(y_hbm.at[i], yv, sem).wait()
        pltpu.make_async_copy(l0_hbm.at[i], lv, sem).wait()
        pltpu.make_async_copy(mask_hbm.at[pl.ds(i * NNODES, NNODES)],
                              mv.at[pl.ds(0, NNODES)], sem).wait()

    def wait_out(ov, sem):
        pltpu.make_async_copy(ov.at[pl.ds(0, NNODES)],
                              out_hbm.at[pl.ds(base * NNODES, NNODES)], sem).wait()

    start_in(base, stops_v0, y_v0, l0_v0, mask_v0, sem_in0)

    def body(j2, carry):
        i0 = base + 2 * j2
        i1 = i0 + 1
        wait_in(i0, stops_v0, y_v0, l0_v0, mask_v0, sem_in0)
        start_in(i1, stops_v1, y_v1, l0_v1, mask_v1, sem_in1)

        @pl.when(j2 >= 1)
        def _():
            wait_out(out_v0, sem_out0)

        _sample_fused(out_hbm, stops_v0, y_v0, l0_v0, mask_v0, sums_v, cnt_v,
                      out_v0, sem_out0, i0)
        wait_in(i1, stops_v1, y_v1, l0_v1, mask_v1, sem_in1)

        @pl.when(2 * j2 + 2 < SPW)
        def _():
            start_in(i0 + 2, stops_v0, y_v0, l0_v0, mask_v0, sem_in0)

        @pl.when(j2 >= 1)
        def _():
            wait_out(out_v1, sem_out1)

        _sample_fused(out_hbm, stops_v1, y_v1, l0_v1, mask_v1, sums_v, cnt_v,
                      out_v1, sem_out1, i1)
        return carry

    lax.fori_loop(0, SPW // 2, body, 0)
    wait_out(out_v0, sem_out0)
    wait_out(out_v1, sem_out1)


def _seg_softmax(stops_padded, yk, l0, x_mask):
    mesh = plsc.VectorSubcoreMesh(
        core_axis_name="c", subcore_axis_name="s", num_cores=NC, num_subcores=NS
    )
    f = pl.kernel(
        _seg_softmax_body,
        out_type=jax.ShapeDtypeStruct((B * NNODES,), jnp.float32),
        mesh=mesh,
        scratch_types=[
            pltpu.VMEM((LPAD,), jnp.int32),
            pltpu.VMEM((LPAD,), jnp.int32),
            pltpu.VMEM((NPAD,), jnp.float32),
            pltpu.VMEM((NPAD,), jnp.float32),
            pltpu.VMEM((NPAD,), jnp.float32),
            pltpu.VMEM((NPAD,), jnp.float32),
            pltpu.VMEM((NPAD,), jnp.int32),
            pltpu.VMEM((NPAD,), jnp.int32),
            pltpu.VMEM((NPAD,), jnp.float32),
            pltpu.VMEM((NPAD,), jnp.float32),
            pltpu.VMEM((NPAD,), jnp.float32),
            pltpu.VMEM((NPAD,), jnp.float32),
            pltpu.SemaphoreType.DMA,
            pltpu.SemaphoreType.DMA,
            pltpu.SemaphoreType.DMA,
            pltpu.SemaphoreType.DMA,
        ],
        compiler_params=pltpu.CompilerParams(needs_layout_passes=False),
    )
    return f(stops_padded, yk, l0, x_mask.reshape(B * NNODES)).reshape(B, NNODES)


def kernel(stops, x, x_dist, x_features, x_week, x_mask, stop_emb_table,
           week_emb_table, W_l, b_l, W_r, W_fc2, b_fc2):
    w = jnp.concatenate([W_l, W_r], axis=1)
    wv = jnp.stack([W_fc2[78, 0], W_fc2[79, 0]])
    dist2d = x_dist.reshape(1, NNODES)
    yk, l0 = _project(x, w, dist2d, wv)
    stops_padded = jnp.pad(stops, ((0, 0), (0, LPAD - L)))
    return _seg_softmax(stops_padded, yk, l0, x_mask)
